# baseline (speedup 1.0000x reference)
import jax
import jax.numpy as jnp
from jax.experimental import pallas as pl

NUM_CATEGORIES = 10
ITEMS_PER_CAT = 10
NUM_ITEMS = NUM_CATEGORIES * ITEMS_PER_CAT
NUM_PARAMS = 64
T_BLOCK = 128


def _nested_logit_block(xc_ref, xi_ref, avail_ref, tc_ref, ti_ref, lam_ref, out_ref):
    tc = tc_ref[0, :]
    ti = ti_ref[0, :]
    lam = lam_ref[0, :]

    W = jnp.sum(xc_ref[...] * tc[None, None, :], axis=-1)
    Y = jnp.sum(xi_ref[...] * ti[None, None, None, :], axis=-1)

    neg_big = jnp.finfo(jnp.float32).min / 2.0
    Y = jnp.where(avail_ref[...] != 0, Y, neg_big)
    Y = Y / lam[None, :, None]

    m = jnp.max(Y, axis=-1)
    s = jnp.sum(jnp.exp(Y - m[..., None]), axis=-1)
    inclusive = m + jnp.log(s)

    logit_cat = W + lam[None, :] * inclusive
    zm = jnp.max(logit_cat, axis=-1, keepdims=True)
    logZ = zm + jnp.log(jnp.sum(jnp.exp(logit_cat - zm), axis=-1, keepdims=True))

    out_ref[...] = (Y - inclusive[..., None]) + (logit_cat - logZ)[..., None]


def kernel(x_category, x_item, user_index, item_availability, theta_category,
           theta_item, lambda_weight):
    del user_index
    T = x_category.shape[0]
    xi4 = x_item.reshape(T, NUM_CATEGORIES, ITEMS_PER_CAT, NUM_PARAMS)
    avail = item_availability.reshape(T, NUM_CATEGORIES, ITEMS_PER_CAT)
    avail = avail.astype(jnp.float32)
    tc2 = theta_category.reshape(1, NUM_PARAMS)
    ti2 = theta_item.reshape(1, NUM_PARAMS)
    lam2 = lambda_weight.reshape(1, NUM_CATEGORIES)

    grid = (T // T_BLOCK,)
    out = pl.pallas_call(
        _nested_logit_block,
        grid=grid,
        in_specs=[
            pl.BlockSpec((T_BLOCK, NUM_CATEGORIES, NUM_PARAMS), lambda i: (i, 0, 0)),
            pl.BlockSpec((T_BLOCK, NUM_CATEGORIES, ITEMS_PER_CAT, NUM_PARAMS),
                         lambda i: (i, 0, 0, 0)),
            pl.BlockSpec((T_BLOCK, NUM_CATEGORIES, ITEMS_PER_CAT), lambda i: (i, 0, 0)),
            pl.BlockSpec((1, NUM_PARAMS), lambda i: (0, 0)),
            pl.BlockSpec((1, NUM_PARAMS), lambda i: (0, 0)),
            pl.BlockSpec((1, NUM_CATEGORIES), lambda i: (0, 0)),
        ],
        out_specs=pl.BlockSpec((T_BLOCK, NUM_CATEGORIES, ITEMS_PER_CAT),
                               lambda i: (i, 0, 0)),
        out_shape=jax.ShapeDtypeStruct((T, NUM_CATEGORIES, ITEMS_PER_CAT),
                                       jnp.float32),
    )(x_category, xi4, avail, tc2, ti2, lam2)
    return out.reshape(T, NUM_ITEMS)


# speedup vs baseline: 1.6150x; 1.6150x over previous
import jax
import jax.numpy as jnp
import numpy as np
from jax.experimental import pallas as pl

NUM_CATEGORIES = 10
ITEMS_PER_CAT = 10
NUM_ITEMS = NUM_CATEGORIES * ITEMS_PER_CAT
NUM_PARAMS = 64
T_BLOCK = 256

_SEG = np.repeat(np.arange(NUM_CATEGORIES), ITEMS_PER_CAT)
_S_NP = (np.arange(NUM_CATEGORIES)[None, :] == _SEG[:, None]).astype(np.float32)


def _nested_logit_block(xc_ref, xi_ref, avail_ref, tc_ref, ti_ref, lam_ref, out_ref):
    tc = tc_ref[0, :]
    ti = ti_ref[0, :]
    lam = lam_ref[...]
    rows = jax.lax.broadcasted_iota(jnp.int32, (NUM_ITEMS, NUM_CATEGORIES), 0)
    cols = jax.lax.broadcasted_iota(jnp.int32, (NUM_ITEMS, NUM_CATEGORIES), 1)
    S = (rows // ITEMS_PER_CAT == cols).astype(jnp.float32)

    W = jnp.sum(xc_ref[...] * tc[None, None, :], axis=-1)
    Y = jnp.sum(xi_ref[...] * ti[None, None, :], axis=-1)

    neg_big = jnp.finfo(jnp.float32).min / 2.0
    Y = jnp.where(avail_ref[...] != 0, Y, neg_big)
    lam_items = jnp.dot(lam, S.T, preferred_element_type=jnp.float32)
    Y = Y / lam_items

    m = jnp.max(Y, axis=-1, keepdims=True)
    e = jnp.exp(Y - m)
    s = jnp.dot(e, S, preferred_element_type=jnp.float32)
    inclusive = m + jnp.log(s)

    logit_cat = W + lam * inclusive
    zm = jnp.max(logit_cat, axis=-1, keepdims=True)
    logZ = zm + jnp.log(jnp.sum(jnp.exp(logit_cat - zm), axis=-1, keepdims=True))

    cat_part = (logit_cat - logZ) - inclusive
    out_ref[...] = Y + jnp.dot(cat_part, S.T, preferred_element_type=jnp.float32)


def kernel(x_category, x_item, user_index, item_availability, theta_category,
           theta_item, lambda_weight):
    del user_index
    T = x_category.shape[0]
    avail = item_availability.astype(jnp.float32)
    tc2 = theta_category.reshape(1, NUM_PARAMS)
    ti2 = theta_item.reshape(1, NUM_PARAMS)
    lam2 = lambda_weight.reshape(1, NUM_CATEGORIES)

    grid = (T // T_BLOCK,)
    out = pl.pallas_call(
        _nested_logit_block,
        grid=grid,
        in_specs=[
            pl.BlockSpec((T_BLOCK, NUM_CATEGORIES, NUM_PARAMS), lambda i: (i, 0, 0)),
            pl.BlockSpec((T_BLOCK, NUM_ITEMS, NUM_PARAMS), lambda i: (i, 0, 0)),
            pl.BlockSpec((T_BLOCK, NUM_ITEMS), lambda i: (i, 0)),
            pl.BlockSpec((1, NUM_PARAMS), lambda i: (0, 0)),
            pl.BlockSpec((1, NUM_PARAMS), lambda i: (0, 0)),
            pl.BlockSpec((1, NUM_CATEGORIES), lambda i: (0, 0)),
        ],
        out_specs=pl.BlockSpec((T_BLOCK, NUM_ITEMS), lambda i: (i, 0)),
        out_shape=jax.ShapeDtypeStruct((T, NUM_ITEMS), jnp.float32),
    )(x_category, x_item, avail, tc2, ti2, lam2)
    return out


# speedup vs baseline: 2.9101x; 1.8019x over previous
import jax
import jax.numpy as jnp
import numpy as np
from jax.experimental import pallas as pl

NUM_CATEGORIES = 10
ITEMS_PER_CAT = 10
NUM_ITEMS = NUM_CATEGORIES * ITEMS_PER_CAT
NUM_PARAMS = 64
LANES = 128
T_BLOCK = 256

_SEG = np.repeat(np.arange(NUM_CATEGORIES), ITEMS_PER_CAT)
_COL_ITEM = np.concatenate([np.arange(NUM_ITEMS),
                            np.arange(LANES - NUM_ITEMS)])

_IND_ITEM = (np.repeat(np.arange(NUM_ITEMS), NUM_PARAMS)[:, None]
             == _COL_ITEM[None, :]).astype(np.float32)
_IND_CAT = (np.repeat(np.arange(NUM_CATEGORIES), NUM_PARAMS)[:, None]
            == np.arange(LANES)[None, :]).astype(np.float32)
_S_SUM = np.zeros((LANES, NUM_CATEGORIES), np.float32)
_S_SUM[np.arange(NUM_ITEMS), _SEG] = 1.0
_S_BCAST = np.zeros((NUM_CATEGORIES, NUM_ITEMS), np.float32)
_S_BCAST[_SEG, np.arange(NUM_ITEMS)] = 1.0


def _nested_logit_block(xc_ref, xi_ref, avail_ref, thi_ref, thc_ref,
                        lam_ref, mval_ref, ssum_ref, sb_ref, out_ref):
    f32 = jnp.float32
    Y = jax.lax.dot(xi_ref[...], thi_ref[...], preferred_element_type=f32)
    W = jax.lax.dot(xc_ref[...], thc_ref[...], preferred_element_type=f32)

    Yv = jnp.where(avail_ref[...] != 0, Y[:, :NUM_ITEMS], mval_ref[...])
    m = jnp.max(Y, axis=-1, keepdims=True)
    e = jnp.exp(Yv - m)
    s = jax.lax.dot(e, ssum_ref[:NUM_ITEMS, :], preferred_element_type=f32)
    inclusive = m + jnp.log(s)

    logit_cat = W[:, :NUM_CATEGORIES] + lam_ref[...] * inclusive
    zm = jnp.max(logit_cat, axis=-1, keepdims=True)
    logZ = zm + jnp.log(jnp.sum(jnp.exp(logit_cat - zm), axis=-1, keepdims=True))

    cat_part = (logit_cat - logZ) - inclusive
    back = jax.lax.dot(cat_part, sb_ref[...], preferred_element_type=f32)
    out_ref[...] = Yv + back


def kernel(x_category, x_item, user_index, item_availability, theta_category,
           theta_item, lambda_weight):
    del user_index
    T = x_category.shape[0]
    xi2 = x_item.reshape(T, NUM_ITEMS * NUM_PARAMS)
    xc2 = x_category.reshape(T, NUM_CATEGORIES * NUM_PARAMS)
    avail = item_availability.astype(jnp.float32)

    inv_lam_item = (1.0 / lambda_weight)[np.asarray(_SEG)]
    inv_lam_col = inv_lam_item[np.asarray(_COL_ITEM)]
    thetas_rep = jnp.tile(theta_item, NUM_ITEMS)
    thi = jnp.asarray(_IND_ITEM) * thetas_rep[:, None] * inv_lam_col[None, :]
    thc = jnp.asarray(_IND_CAT) * jnp.tile(theta_category, NUM_CATEGORIES)[:, None]
    neg_big = float(np.finfo(np.float32).min / 2.0)
    mval = (neg_big * inv_lam_item).reshape(1, NUM_ITEMS)
    lam2 = lambda_weight.reshape(1, NUM_CATEGORIES)

    grid = (T // T_BLOCK,)
    const = lambda i: (0, 0)
    out = pl.pallas_call(
        _nested_logit_block,
        grid=grid,
        in_specs=[
            pl.BlockSpec((T_BLOCK, NUM_CATEGORIES * NUM_PARAMS), lambda i: (i, 0)),
            pl.BlockSpec((T_BLOCK, NUM_ITEMS * NUM_PARAMS), lambda i: (i, 0)),
            pl.BlockSpec((T_BLOCK, NUM_ITEMS), lambda i: (i, 0)),
            pl.BlockSpec((NUM_ITEMS * NUM_PARAMS, LANES), const),
            pl.BlockSpec((NUM_CATEGORIES * NUM_PARAMS, LANES), const),
            pl.BlockSpec((1, NUM_CATEGORIES), const),
            pl.BlockSpec((1, NUM_ITEMS), const),
            pl.BlockSpec((LANES, NUM_CATEGORIES), const),
            pl.BlockSpec((NUM_CATEGORIES, NUM_ITEMS), const),
        ],
        out_specs=pl.BlockSpec((T_BLOCK, NUM_ITEMS), lambda i: (i, 0)),
        out_shape=jax.ShapeDtypeStruct((T, NUM_ITEMS), jnp.float32),
    )(xc2, xi2, avail, thi, thc, lam2, mval,
      jnp.asarray(_S_SUM), jnp.asarray(_S_BCAST))
    return out
